# Initial kernel scaffold; baseline (speedup 1.0000x reference)
import jax
import jax.numpy as jnp
from jax.experimental import pallas as pl


def kernel(timestamps, hour_table, weekday_table, month_table, season_table, W, b):
    raise NotImplementedError("write your pallas kernel here")


# speedup vs baseline: 14.1877x; 14.1877x over previous
import functools

import jax
import jax.numpy as jnp
from jax import lax
from jax.experimental import pallas as pl
from jax.experimental.pallas import tpu as pltpu
from jax.experimental.pallas import tpu_sc as plsc

B, S, D = 4096, 200, 64
DQ = D // 4
N_TOK = B * S
N_ROWS = 24 * 7 * 12

NC, NS = 2, 16
NW = NC * NS
TOK_PER_W = N_TOK // NW
CHUNK = 512
N_CHUNKS = TOK_PER_W // CHUNK


def _table_body(hour_ref, week_ref, month_ref, season_ref, w_ref, b_ref, ft_ref):
    w = w_ref[...]
    ht = jnp.dot(hour_ref[...], w[:, 0:DQ].T, preferred_element_type=jnp.float32)
    wt = jnp.dot(week_ref[...], w[:, DQ:2 * DQ].T, preferred_element_type=jnp.float32)
    mt = jnp.dot(month_ref[...], w[:, 2 * DQ:3 * DQ].T, preferred_element_type=jnp.float32)
    st = jnp.dot(season_ref[...], w[:, 3 * DQ:4 * DQ].T, preferred_element_type=jnp.float32)
    s_oh = (lax.broadcasted_iota(jnp.int32, (12, 4), 0) // 3
            == lax.broadcasted_iota(jnp.int32, (12, 4), 1)).astype(jnp.float32)
    mt2 = mt + jnp.dot(s_oh, st, preferred_element_type=jnp.float32) + b_ref[...][None, :]
    c_h = lax.broadcasted_iota(jnp.int32, (N_ROWS, 24), 0) // 84
    oh_h = (c_h == lax.broadcasted_iota(jnp.int32, (N_ROWS, 24), 1)).astype(jnp.float32)
    c_w = (lax.broadcasted_iota(jnp.int32, (N_ROWS, 7), 0) // 12) % 7
    oh_w = (c_w == lax.broadcasted_iota(jnp.int32, (N_ROWS, 7), 1)).astype(jnp.float32)
    c_m = lax.broadcasted_iota(jnp.int32, (N_ROWS, 12), 0) % 12
    oh_m = (c_m == lax.broadcasted_iota(jnp.int32, (N_ROWS, 12), 1)).astype(jnp.float32)
    ft_ref[...] = (jnp.dot(oh_h, ht, preferred_element_type=jnp.float32)
                   + jnp.dot(oh_w, wt, preferred_element_type=jnp.float32)
                   + jnp.dot(oh_m, mt2, preferred_element_type=jnp.float32))


def _build_table(hour_table, weekday_table, month_table, season_table, w, b):
    return pl.pallas_call(
        _table_body,
        out_shape=jax.ShapeDtypeStruct((N_ROWS, D), jnp.float32),
    )(hour_table, weekday_table, month_table, season_table, w, b)


def _idx_body(ts_ref, idx_ref):
    t = ts_ref[...]
    h = (t // 60) % 24
    wd = (t // 1440) % 7
    m = (t // 43200) % 12
    idx_ref[...] = h * 84 + wd * 12 + m


def _build_idx(timestamps):
    blk = 512
    return pl.pallas_call(
        _idx_body,
        grid=(B // blk,),
        in_specs=[pl.BlockSpec((blk, S), lambda i: (i, 0))],
        out_specs=pl.BlockSpec((blk, S), lambda i: (i, 0)),
        out_shape=jax.ShapeDtypeStruct((B, S), jnp.int32),
    )(timestamps)


def _sc_gather_body(ft_hbm, idx_hbm, out_hbm, idx_v, rows_v, sem):
    wid = lax.axis_index("s") * NC + lax.axis_index("c")
    base0 = wid * TOK_PER_W

    def body(i, _):
        base = base0 + i * CHUNK
        pltpu.sync_copy(idx_hbm.at[pl.ds(base, CHUNK)], idx_v)
        pltpu.async_copy(ft_hbm.at[idx_v], rows_v, sem).wait()
        pltpu.sync_copy(rows_v, out_hbm.at[pl.ds(base, CHUNK)])
        return 0

    lax.fori_loop(0, N_CHUNKS, body, 0)


@functools.cache
def _sc_gather():
    return functools.partial(
        pl.kernel,
        mesh=plsc.VectorSubcoreMesh(core_axis_name="c", subcore_axis_name="s"),
        out_type=jax.ShapeDtypeStruct((N_TOK, D), jnp.float32),
        scratch_types=[
            pltpu.VMEM((CHUNK,), jnp.int32),
            pltpu.VMEM((CHUNK, D), jnp.float32),
            pltpu.SemaphoreType.DMA,
        ],
        compiler_params=pltpu.CompilerParams(use_tc_tiling_on_sc=False),
    )(_sc_gather_body)


def kernel(timestamps, hour_table, weekday_table, month_table, season_table, W, b):
    ft = _build_table(hour_table, weekday_table, month_table, season_table, W, b)
    idx = _build_idx(timestamps).reshape(N_TOK)
    out = _sc_gather()(ft, idx)
    return out.reshape(B, S, D)


# speedup vs baseline: 14.2539x; 1.0047x over previous
import functools

import jax
import jax.numpy as jnp
from jax import lax
from jax.experimental import pallas as pl
from jax.experimental.pallas import tpu as pltpu
from jax.experimental.pallas import tpu_sc as plsc

B, S, D = 4096, 200, 64
DQ = D // 4
N_TOK = B * S
N_ROWS = 24 * 7 * 12

NC, NS = 2, 16
NW = NC * NS
ROWS_PER_W = B // NW
R_CHUNK = 4
CHUNK = R_CHUNK * S
N_CHUNKS = ROWS_PER_W // R_CHUNK


def _table_body(hour_ref, week_ref, month_ref, season_ref, w_ref, b_ref, ft_ref):
    w = w_ref[...]
    ht = jnp.dot(hour_ref[...], w[:, 0:DQ].T, preferred_element_type=jnp.float32)
    wt = jnp.dot(week_ref[...], w[:, DQ:2 * DQ].T, preferred_element_type=jnp.float32)
    mt = jnp.dot(month_ref[...], w[:, 2 * DQ:3 * DQ].T, preferred_element_type=jnp.float32)
    st = jnp.dot(season_ref[...], w[:, 3 * DQ:4 * DQ].T, preferred_element_type=jnp.float32)
    s_oh = (lax.broadcasted_iota(jnp.int32, (12, 4), 0) // 3
            == lax.broadcasted_iota(jnp.int32, (12, 4), 1)).astype(jnp.float32)
    mt2 = mt + jnp.dot(s_oh, st, preferred_element_type=jnp.float32) + b_ref[...][None, :]
    c_h = lax.broadcasted_iota(jnp.int32, (N_ROWS, 24), 0) // 84
    oh_h = (c_h == lax.broadcasted_iota(jnp.int32, (N_ROWS, 24), 1)).astype(jnp.float32)
    c_w = (lax.broadcasted_iota(jnp.int32, (N_ROWS, 7), 0) // 12) % 7
    oh_w = (c_w == lax.broadcasted_iota(jnp.int32, (N_ROWS, 7), 1)).astype(jnp.float32)
    c_m = lax.broadcasted_iota(jnp.int32, (N_ROWS, 12), 0) % 12
    oh_m = (c_m == lax.broadcasted_iota(jnp.int32, (N_ROWS, 12), 1)).astype(jnp.float32)
    ft_ref[...] = (jnp.dot(oh_h, ht, preferred_element_type=jnp.float32)
                   + jnp.dot(oh_w, wt, preferred_element_type=jnp.float32)
                   + jnp.dot(oh_m, mt2, preferred_element_type=jnp.float32))


def _build_table(hour_table, weekday_table, month_table, season_table, w, b):
    return pl.pallas_call(
        _table_body,
        out_shape=jax.ShapeDtypeStruct((N_ROWS, D), jnp.float32),
    )(hour_table, weekday_table, month_table, season_table, w, b)


def _idx_body(ts_ref, idx_ref):
    t = ts_ref[...]
    h = (t // 60) % 24
    wd = (t // 1440) % 7
    m = (t // 43200) % 12
    idx_ref[...] = h * 84 + wd * 12 + m


def _build_idx(timestamps):
    blk = 512
    return pl.pallas_call(
        _idx_body,
        grid=(B // blk,),
        in_specs=[pl.BlockSpec((blk, S), lambda i: (i, 0))],
        out_specs=pl.BlockSpec((blk, S), lambda i: (i, 0)),
        out_shape=jax.ShapeDtypeStruct((B, S), jnp.int32),
    )(timestamps)


def _sc_gather_body(ft_hbm, idx_hbm, out_hbm, idx_v, rows_v, sem):
    wid = lax.axis_index("s") * NC + lax.axis_index("c")
    row0 = wid * ROWS_PER_W

    def body(i, _):
        base_row = row0 + i * R_CHUNK
        pltpu.sync_copy(idx_hbm.at[pl.ds(base_row * S, CHUNK)], idx_v)
        pltpu.async_copy(ft_hbm.at[idx_v], rows_v, sem).wait()
        for r in range(R_CHUNK):
            pltpu.sync_copy(rows_v.at[pl.ds(r * S, S)], out_hbm.at[base_row + r])
        return 0

    lax.fori_loop(0, N_CHUNKS, body, 0)


@functools.cache
def _sc_gather():
    return functools.partial(
        pl.kernel,
        mesh=plsc.VectorSubcoreMesh(core_axis_name="c", subcore_axis_name="s"),
        out_type=jax.ShapeDtypeStruct((B, S, D), jnp.float32),
        scratch_types=[
            pltpu.VMEM((CHUNK,), jnp.int32),
            pltpu.VMEM((CHUNK, D), jnp.float32),
            pltpu.SemaphoreType.DMA,
        ],
        compiler_params=pltpu.CompilerParams(use_tc_tiling_on_sc=False),
    )(_sc_gather_body)


def kernel(timestamps, hour_table, weekday_table, month_table, season_table, W, b):
    ft = _build_table(hour_table, weekday_table, month_table, season_table, W, b)
    idx = _build_idx(timestamps).reshape(N_TOK)
    return _sc_gather()(ft, idx)
